# baseline (speedup 1.0000x reference)
import jax
import jax.numpy as jnp
from jax import lax
from jax.experimental import pallas as pl
from jax.experimental.pallas import tpu as pltpu
from jax.experimental.pallas import tpu_sc as plsc

B = 1024
K = 1024
D = 64
L = 16
NC = 2
NS = 16
NW = NC * NS
RPW = B // NW


def _sc_body(cb_hbm, logits_hbm, out_hbm,
             logits_v, vmax_b, vidx_b, idx_v, rows_v, sem):
    w = lax.axis_index("s") * NC + lax.axis_index("c")
    base = w * RPW
    pltpu.sync_copy(logits_hbm.at[pl.ds(base, RPW)], logits_v)
    lane = lax.iota(jnp.int32, L)

    def row_scan(r, carry):
        def chunk(c, st):
            vmax, vidx, col = st
            v = logits_v[r, pl.ds(c * L, L)]
            pred = v > vmax
            return (jnp.where(pred, v, vmax),
                    jnp.where(pred, col, vidx),
                    col + L)

        vmax, vidx, _ = lax.fori_loop(
            0, K // L, chunk,
            (jnp.full((L,), -jnp.inf, jnp.float32),
             jnp.zeros((L,), jnp.int32),
             lane))
        vmax_b[pl.ds(r * L, L)] = vmax
        vidx_b[pl.ds(r * L, L)] = vidx
        return carry

    lax.fori_loop(0, RPW, row_scan, 0)

    def group(g, carry):
        rows = g * L + lane

        def slot(j, st):
            bmax, bidx = st
            jv = rows * L + j
            v = plsc.load_gather(vmax_b, [jv])
            i = plsc.load_gather(vidx_b, [jv])
            pred = (v > bmax) | ((v == bmax) & (i < bidx))
            return (jnp.where(pred, v, bmax), jnp.where(pred, i, bidx))

        bmax, bidx = lax.fori_loop(
            0, L, slot,
            (jnp.full((L,), -jnp.inf, jnp.float32),
             jnp.full((L,), jnp.int32(2**30))))
        idx_v[pl.ds(g * L, L)] = (base + rows) * K + bidx
        return carry

    lax.fori_loop(0, RPW // L, group, 0)

    pltpu.async_copy(cb_hbm.at[idx_v], rows_v, sem).wait()
    pltpu.sync_copy(rows_v, out_hbm.at[pl.ds(base, RPW)])


_sc_call = pl.kernel(
    _sc_body,
    out_type=jax.ShapeDtypeStruct((B, D), jnp.float32),
    mesh=plsc.VectorSubcoreMesh(core_axis_name="c", subcore_axis_name="s"),
    compiler_params=pltpu.CompilerParams(
        needs_layout_passes=False, use_tc_tiling_on_sc=False),
    scratch_types=[
        pltpu.VMEM((RPW, K), jnp.float32),
        pltpu.VMEM((RPW * L,), jnp.float32),
        pltpu.VMEM((RPW * L,), jnp.int32),
        pltpu.VMEM((RPW,), jnp.int32),
        pltpu.VMEM((RPW, D), jnp.float32),
        pltpu.SemaphoreType.DMA,
    ],
)


def kernel(subCodebook, logits):
    return _sc_call(subCodebook.reshape(B * K, D), logits)


# speedup vs baseline: 10.7101x; 10.7101x over previous
import jax
import jax.numpy as jnp
from jax import lax
from jax.experimental import pallas as pl
from jax.experimental.pallas import tpu as pltpu
from jax.experimental.pallas import tpu_sc as plsc

B = 1024
K = 1024
D = 64
L = 16
NC = 2
NS = 16
NW = NC * NS
RPW = B // NW
VR = B * K * D // 128
NB = RPW * D // 128


def _sc_body(cb_hbm, logits_hbm, out_hbm,
             logits_v, vmax_b, vidx_b, idx_v, idx_b, gbuf, outv, sem):
    w = lax.axis_index("s") * NC + lax.axis_index("c")
    base = w * RPW
    pltpu.sync_copy(logits_hbm.at[pl.ds(base, RPW)], logits_v)
    lane = lax.iota(jnp.int32, L)

    def row_scan(r, carry):
        def chunk(c, st):
            vmax, vidx, col = st
            v = logits_v[r, pl.ds(c * L, L)]
            pred = v > vmax
            return (jnp.where(pred, v, vmax),
                    jnp.where(pred, col, vidx),
                    col + L)

        vmax, vidx, _ = lax.fori_loop(
            0, K // L, chunk,
            (jnp.full((L,), -jnp.inf, jnp.float32),
             jnp.zeros((L,), jnp.int32),
             lane))
        vmax_b[pl.ds(r * L, L)] = vmax
        vidx_b[pl.ds(r * L, L)] = vidx
        return carry

    lax.fori_loop(0, RPW, row_scan, 0)

    def group(g, carry):
        rows = g * L + lane

        def slot(j, st):
            bmax, bidx = st
            jv = rows * L + j
            v = plsc.load_gather(vmax_b, [jv])
            i = plsc.load_gather(vidx_b, [jv])
            pred = (v > bmax) | ((v == bmax) & (i < bidx))
            return (jnp.where(pred, v, bmax), jnp.where(pred, i, bidx))

        _, bidx = lax.fori_loop(
            0, L, slot,
            (jnp.full((L,), -jnp.inf, jnp.float32),
             jnp.full((L,), jnp.int32(2**30))))
        idx_v[pl.ds(g * L, L)] = bidx
        bk = (base + rows) * (VR // B) + ((bidx >> 7) << 3)

        def dloop(d, c2):
            e = rows * D + d
            val = bk + (((d >> 3) << 6) + (d & 7))
            plsc.store_scatter(idx_b, [e >> 7, e & 127], val)
            return c2

        lax.fori_loop(0, D, dloop, 0)
        return carry

    lax.fori_loop(0, RPW // L, group, 0)

    for g in range(RPW // L):
        kv = idx_v[pl.ds(g * L, L)]
        for s in range(NB // 2):
            sb = g * (NB // 2) + s
            pltpu.async_copy(cb_hbm.at[idx_b.at[sb]], gbuf, sem).wait()
            for rl in range(2):
                r = sb * 2 + rl
                klo = kv[s * 2 + rl] & 127
                for c in range(D // L):
                    rowi = rl * D + c * L + lane
                    v = plsc.load_gather(gbuf, [rowi, jnp.full((L,), klo)])
                    outv[r, pl.ds(c * L, L)] = v

    pltpu.sync_copy(outv, out_hbm.at[pl.ds(base, RPW)])


_sc_call = pl.kernel(
    _sc_body,
    out_type=jax.ShapeDtypeStruct((B, D), jnp.float32),
    mesh=plsc.VectorSubcoreMesh(core_axis_name="c", subcore_axis_name="s"),
    compiler_params=pltpu.CompilerParams(
        needs_layout_passes=False, use_tc_tiling_on_sc=True),
    scratch_types=[
        pltpu.VMEM((RPW, K), jnp.float32),
        pltpu.VMEM((RPW * L,), jnp.float32),
        pltpu.VMEM((RPW * L,), jnp.int32),
        pltpu.VMEM((RPW,), jnp.int32),
        pltpu.VMEM((NB, 128), jnp.int32),
        pltpu.VMEM((128, 128), jnp.float32),
        pltpu.VMEM((RPW, D), jnp.float32),
        pltpu.SemaphoreType.DMA,
    ],
)


def kernel(subCodebook, logits):
    cb5 = subCodebook.reshape(B, K // 128, 128, D // 8, 8)
    cb5 = cb5.transpose(0, 3, 1, 4, 2)
    return _sc_call(cb5.reshape(VR, 128), logits)


# speedup vs baseline: 18.9228x; 1.7668x over previous
import jax
import jax.numpy as jnp
from jax import lax
from jax.experimental import pallas as pl
from jax.experimental.pallas import tpu as pltpu
from jax.experimental.pallas import tpu_sc as plsc

B = 1024
K = 1024
D = 64
L = 16
NC = 2
NS = 16
NW = NC * NS
RPW = B // NW
LV = RPW * K
NB = RPW * D // 128


def _sc_body(cb_hbm, logits_hbm, out_hbm,
             logits_v, vmax_b, vidx_b, idx_v, idx_b, g8, outv, sem):
    w = lax.axis_index("s") * NC + lax.axis_index("c")
    base = w * RPW
    pltpu.sync_copy(logits_hbm.at[pl.ds(w * LV, LV)], logits_v)
    lane = lax.iota(jnp.int32, L)

    def row_scan(p, carry):
        r = p * 2
        a0 = (r >> 3) * (K * 8) + (r & 7) * 128

        def chunk(tq, st):
            xm, xi, ym, yi = st
            adr = a0 + (tq >> 3) * 1024 + (tq & 7) * L
            col = tq * L + lane
            va = logits_v[pl.ds(adr, L)]
            vb = logits_v[pl.ds(adr + 128, L)]
            pa = va > xm
            pb = vb > ym
            return (jnp.where(pa, va, xm), jnp.where(pa, col, xi),
                    jnp.where(pb, vb, ym), jnp.where(pb, col, yi))

        ninf = jnp.full((L,), -jnp.inf, jnp.float32)
        zi = jnp.zeros((L,), jnp.int32)
        xm, xi, ym, yi = lax.fori_loop(
            0, K // L, chunk, (ninf, zi, ninf, zi))
        vmax_b[pl.ds(r * L, L)] = xm
        vidx_b[pl.ds(r * L, L)] = xi
        vmax_b[pl.ds(r * L + L, L)] = ym
        vidx_b[pl.ds(r * L + L, L)] = yi
        return carry

    lax.fori_loop(0, RPW // 2, row_scan, 0)

    def group(g, carry):
        rows = g * L + lane

        def slot(j, st):
            bmax, bidx = st
            jv = rows * L + j
            v = plsc.load_gather(vmax_b, [jv])
            i = plsc.load_gather(vidx_b, [jv])
            pred = (v > bmax) | ((v == bmax) & (i < bidx))
            return (jnp.where(pred, v, bmax), jnp.where(pred, i, bidx))

        _, bidx = lax.fori_loop(
            0, L, slot,
            (jnp.full((L,), -jnp.inf, jnp.float32),
             jnp.full((L,), jnp.int32(2**30))))
        idx_v[pl.ds(g * L, L)] = bidx
        bk = ((base + rows) * (K * D // 8)
              + ((bidx >> 7) << 7) + ((bidx & 127) >> 3))

        def dloop(d, c2):
            e = rows * D + d
            val = bk + (((d >> 3) << 10) + ((d & 7) << 4))
            plsc.store_scatter(idx_b, [e >> 7, e & 127], val)
            return c2

        lax.fori_loop(0, D, dloop, 0)
        return carry

    lax.fori_loop(0, RPW // L, group, 0)

    copies = [
        pltpu.async_copy(cb_hbm.at[idx_b.at[s]],
                         g8.at[pl.ds(s * 128, 128)], sem)
        for s in range(NB)
    ]
    for c in copies:
        c.wait()

    for g in range(RPW // L):
        kv = idx_v[pl.ds(g * L, L)]
        for j in range(L):
            r = g * L + j
            k7 = kv[j] & 7
            for c in range(D // L):
                ev = r * D + c * L + lane
                v = plsc.load_gather(g8, [ev, jnp.full((L,), k7)])
                outv[r, pl.ds(c * L, L)] = v

    pltpu.sync_copy(outv, out_hbm.at[pl.ds(base, RPW)])


_sc_call = pl.kernel(
    _sc_body,
    out_type=jax.ShapeDtypeStruct((B, D), jnp.float32),
    mesh=plsc.VectorSubcoreMesh(core_axis_name="c", subcore_axis_name="s"),
    compiler_params=pltpu.CompilerParams(
        needs_layout_passes=False, use_tc_tiling_on_sc=False),
    scratch_types=[
        pltpu.VMEM((LV,), jnp.float32),
        pltpu.VMEM((RPW * L,), jnp.float32),
        pltpu.VMEM((RPW * L,), jnp.int32),
        pltpu.VMEM((RPW,), jnp.int32),
        pltpu.VMEM((NB, 128), jnp.int32),
        pltpu.VMEM((RPW * D, 8), jnp.float32),
        pltpu.VMEM((RPW, D), jnp.float32),
        pltpu.SemaphoreType.DMA,
    ],
)


def kernel(subCodebook, logits):
    cb5 = subCodebook.reshape(B, K // 128, 128, D // 8, 8)
    cb8 = cb5.transpose(0, 3, 1, 4, 2).reshape(B * K * D // 8, 8)
    lg4 = logits.reshape(B // 8, 8, K // 128, 128)
    lgf = lg4.transpose(0, 2, 1, 3).reshape(B * K)
    return _sc_call(cb8, lgf)


# speedup vs baseline: 20.9809x; 1.1088x over previous
import jax
import jax.numpy as jnp
from jax import lax
from jax.experimental import pallas as pl
from jax.experimental.pallas import tpu as pltpu
from jax.experimental.pallas import tpu_sc as plsc

B = 1024
K = 1024
D = 64
L = 16
NC = 2
NS = 16
NW = NC * NS
RPW = B // NW
LV = RPW * K
NB = RPW * D // 128


def _sc_body(cb_hbm, logits_hbm, out_hbm,
             logits_v, vmax_b, vidx_b, idx_b, g8, outv, sem):
    w = lax.axis_index("s") * NC + lax.axis_index("c")
    base = w * RPW
    pltpu.sync_copy(logits_hbm.at[pl.ds(w * LV, LV)], logits_v)
    lane = lax.iota(jnp.int32, L)
    ninf = jnp.full((L,), -jnp.inf, jnp.float32)
    zi = jnp.zeros((L,), jnp.int32)
    dconst = [((c * L + lane) >> 3 << 10) + (((c * L + lane) & 7) << 4)
              for c in range(D // L)]

    copies = []
    kvecs = []
    for g in range(RPW // L):
        def row_scan(p, carry, g=g):
            r = g * L + p * 4
            a0 = (r >> 3) * (K * 8) + (r & 7) * 128

            def chunk(tq, st):
                m0, i0, m1, i1, m2, i2, m3, i3 = st
                adr = a0 + (tq >> 3) * 1024 + (tq & 7) * L
                col = tq * L + lane
                v0 = logits_v[pl.ds(adr, L)]
                v1 = logits_v[pl.ds(adr + 128, L)]
                v2 = logits_v[pl.ds(adr + 256, L)]
                v3 = logits_v[pl.ds(adr + 384, L)]
                p0 = v0 > m0
                p1 = v1 > m1
                p2 = v2 > m2
                p3 = v3 > m3
                return (jnp.where(p0, v0, m0), jnp.where(p0, col, i0),
                        jnp.where(p1, v1, m1), jnp.where(p1, col, i1),
                        jnp.where(p2, v2, m2), jnp.where(p2, col, i2),
                        jnp.where(p3, v3, m3), jnp.where(p3, col, i3))

            st = lax.fori_loop(0, K // L, chunk,
                               (ninf, zi, ninf, zi, ninf, zi, ninf, zi))
            for q in range(4):
                vmax_b[pl.ds((r + q) * L, L)] = st[2 * q]
                vidx_b[pl.ds((r + q) * L, L)] = st[2 * q + 1]
            return carry

        lax.fori_loop(0, L // 4, row_scan, 0)

        rows = g * L + lane

        def slot(j, st):
            bmax, bidx = st
            jv = rows * L + j
            v = plsc.load_gather(vmax_b, [jv])
            i = plsc.load_gather(vidx_b, [jv])
            pred = (v > bmax) | ((v == bmax) & (i < bidx))
            return (jnp.where(pred, v, bmax), jnp.where(pred, i, bidx))

        _, bidx = lax.fori_loop(
            0, L, slot, (ninf, jnp.full((L,), jnp.int32(2**30))))
        kvecs.append(bidx)
        bk = ((base + rows) * (K * D // 8)
              + ((bidx >> 7) << 7) + ((bidx & 127) >> 3))

        for j in range(L):
            r = g * L + j
            bkr = bk[j]
            for c in range(D // L):
                idx_b[r >> 1, pl.ds((r & 1) * D + c * L, L)] = bkr + dconst[c]

        for s in range(g * NB // 2, (g + 1) * NB // 2):
            copies.append(pltpu.async_copy(
                cb_hbm.at[idx_b.at[s]],
                g8.at[pl.ds(s * 128, 128)], sem))

    for c in copies:
        c.wait()

    for g in range(RPW // L):
        kv = kvecs[g]
        for j in range(L):
            r = g * L + j
            k7 = kv[j] & 7
            for c in range(D // L):
                ev = r * D + c * L + lane
                v = plsc.load_gather(g8, [ev, jnp.full((L,), k7)])
                outv[r, pl.ds(c * L, L)] = v

    pltpu.sync_copy(outv, out_hbm.at[pl.ds(base, RPW)])


_sc_call = pl.kernel(
    _sc_body,
    out_type=jax.ShapeDtypeStruct((B, D), jnp.float32),
    mesh=plsc.VectorSubcoreMesh(core_axis_name="c", subcore_axis_name="s"),
    compiler_params=pltpu.CompilerParams(
        needs_layout_passes=False, use_tc_tiling_on_sc=False),
    scratch_types=[
        pltpu.VMEM((LV,), jnp.float32),
        pltpu.VMEM((RPW * L,), jnp.float32),
        pltpu.VMEM((RPW * L,), jnp.int32),
        pltpu.VMEM((NB, 128), jnp.int32),
        pltpu.VMEM((RPW * D, 8), jnp.float32),
        pltpu.VMEM((RPW, D), jnp.float32),
        pltpu.SemaphoreType.DMA,
    ],
)


def kernel(subCodebook, logits):
    cb5 = subCodebook.reshape(B, K // 128, 128, D // 8, 8)
    cb8 = cb5.transpose(0, 3, 1, 4, 2).reshape(B * K * D // 8, 8)
    lg4 = logits.reshape(B // 8, 8, K // 128, 128)
    lgf = lg4.transpose(0, 2, 1, 3).reshape(B * K)
    return _sc_call(cb8, lgf)


# speedup vs baseline: 22.9211x; 1.0925x over previous
import jax
import jax.numpy as jnp
from jax import lax
from jax.experimental import pallas as pl
from jax.experimental.pallas import tpu as pltpu
from jax.experimental.pallas import tpu_sc as plsc

B = 1024
K = 1024
D = 64
L = 16
NC = 2
NS = 16
NW = NC * NS
RPW = B // NW
LV = RPW * K
NB = RPW * D // 128
PS = 17


def _sc_body(cb_hbm, logits_hbm, out_hbm,
             logits_v, vmax_b, vidx_b, idx_b, g8, outn, sem, s0, s1):
    w = lax.axis_index("s") * NC + lax.axis_index("c")
    base = w * RPW
    half = [
        pltpu.async_copy(logits_hbm.at[pl.ds(w * LV + h * (LV // 2), LV // 2)],
                         logits_v.at[pl.ds(h * (LV // 2), LV // 2)],
                         [s0, s1][h])
        for h in range(2)
    ]
    lane = lax.iota(jnp.int32, L)
    ninf = jnp.full((L,), -jnp.inf, jnp.float32)
    zi = jnp.zeros((L,), jnp.int32)
    dconst = [((c * L + lane) >> 3 << 10) + (((c * L + lane) & 7) << 4)
              for c in range(D // L)]

    copies = []
    kvecs = []
    for g in range(RPW // L):
        half[g].wait()

        def row_scan(p, carry, g=g):
            r = g * L + p * 4
            a0 = (r >> 3) * (K * 8) + (r & 7) * 128

            def chunk(tq, st):
                m0, i0, m1, i1, m2, i2, m3, i3 = st
                adr = a0 + (tq >> 3) * 1024 + (tq & 7) * L
                col = tq * L + lane
                v0 = logits_v[pl.ds(adr, L)]
                v1 = logits_v[pl.ds(adr + 128, L)]
                v2 = logits_v[pl.ds(adr + 256, L)]
                v3 = logits_v[pl.ds(adr + 384, L)]
                p0 = v0 > m0
                p1 = v1 > m1
                p2 = v2 > m2
                p3 = v3 > m3
                return (jnp.where(p0, v0, m0), jnp.where(p0, col, i0),
                        jnp.where(p1, v1, m1), jnp.where(p1, col, i1),
                        jnp.where(p2, v2, m2), jnp.where(p2, col, i2),
                        jnp.where(p3, v3, m3), jnp.where(p3, col, i3))

            st = lax.fori_loop(0, K // L, chunk,
                               (ninf, zi, ninf, zi, ninf, zi, ninf, zi))
            for q in range(4):
                vmax_b[pl.ds((r + q) * PS, L)] = st[2 * q]
                vidx_b[pl.ds((r + q) * PS, L)] = st[2 * q + 1]
            return carry

        lax.fori_loop(0, L // 4, row_scan, 0)

        rows = g * L + lane

        def slot(j, st):
            bmax, bidx = st
            jv = rows * PS + j
            v = plsc.load_gather(vmax_b, [jv])
            i = plsc.load_gather(vidx_b, [jv])
            pred = (v > bmax) | ((v == bmax) & (i < bidx))
            return (jnp.where(pred, v, bmax), jnp.where(pred, i, bidx))

        _, bidx = lax.fori_loop(
            0, L, slot, (ninf, jnp.full((L,), jnp.int32(2**30))))
        kvecs.append(bidx)
        bk = ((base + rows) * (K * D // 8)
              + ((bidx >> 7) << 7) + ((bidx & 127) >> 3))

        for j in range(L):
            r = g * L + j
            bkr = bk[j]
            for c in range(D // L):
                idx_b[r >> 1, pl.ds((r & 1) * D + c * L, L)] = bkr + dconst[c]

        for s in range(g * NB // 2, (g + 1) * NB // 2):
            copies.append(pltpu.async_copy(
                cb_hbm.at[idx_b.at[s]],
                g8.at[pl.ds(s * 128, 128)], sem))

    for g in range(RPW // L):
        for c in copies[g * NB // 2:(g + 1) * NB // 2]:
            c.wait()
        k7 = kvecs[g] & 7
        ev0 = (g * L + lane) * D
        for d in range(D):
            v = plsc.load_gather(g8, [ev0 + d, k7])
            outn[d >> 3, d & 7, pl.ds(g * L, L)] = v

    bt = base >> 7
    bl0 = pl.multiple_of(base & 127, RPW)
    pltpu.sync_copy(outn, out_hbm.at[:, bt, :, pl.ds(bl0, RPW)])


_sc_call = pl.kernel(
    _sc_body,
    out_type=jax.ShapeDtypeStruct((D // 8, B // 128, 8, 128), jnp.float32),
    mesh=plsc.VectorSubcoreMesh(core_axis_name="c", subcore_axis_name="s"),
    compiler_params=pltpu.CompilerParams(
        needs_layout_passes=False, use_tc_tiling_on_sc=False),
    scratch_types=[
        pltpu.VMEM((LV,), jnp.float32),
        pltpu.VMEM((RPW * PS,), jnp.float32),
        pltpu.VMEM((RPW * PS,), jnp.int32),
        pltpu.VMEM((NB, 128), jnp.int32),
        pltpu.VMEM((RPW * D, 8), jnp.float32),
        pltpu.VMEM((D // 8, 8, RPW), jnp.float32),
        pltpu.SemaphoreType.DMA,
        pltpu.SemaphoreType.DMA,
        pltpu.SemaphoreType.DMA,
    ],
)


def kernel(subCodebook, logits):
    cb5 = subCodebook.reshape(B, K // 128, 128, D // 8, 8)
    cb8 = cb5.transpose(0, 3, 1, 4, 2).reshape(B * K * D // 8, 8)
    lg4 = logits.reshape(B // 8, 8, K // 128, 128)
    lgf = lg4.transpose(0, 2, 1, 3).reshape(B * K)
    out4 = _sc_call(cb8, lgf)
    return out4.transpose(1, 3, 0, 2).reshape(B, D)
